# baseline (speedup 1.0000x reference)
import functools

import jax
import jax.numpy as jnp
from jax.experimental import pallas as pl


def _layer0_body(adj_ref, xblk_ref, feats_ref, wa_ref, wb_ref,
                 h_ref, deg_ref):
    d = wa_ref.shape[0]
    a16 = adj_ref[...].astype(jnp.bfloat16)
    p_ext = jnp.dot(a16, feats_ref[...], preferred_element_type=jnp.float32)
    deg = p_ext[:, d:d + 1] + 1.0
    neigh = (p_ext[:, :d] / deg).astype(jnp.bfloat16)
    out = (jnp.dot(xblk_ref[...], wa_ref[...], preferred_element_type=jnp.float32)
           + jnp.dot(neigh, wb_ref[...], preferred_element_type=jnp.float32))
    h_ref[...] = jnp.maximum(out, 0.0).astype(jnp.bfloat16)
    deg_ref[...] = deg


def _layer1_body(adj_ref, xblk_ref, feats_ref, wa_ref, wb_ref, deg_ref,
                 out_ref):
    a16 = adj_ref[...].astype(jnp.bfloat16)
    p = jnp.dot(a16, feats_ref[...], preferred_element_type=jnp.float32)
    neigh = (p / deg_ref[...]).astype(jnp.bfloat16)
    out_ref[...] = (
        jnp.dot(xblk_ref[...], wa_ref[...], preferred_element_type=jnp.float32)
        + jnp.dot(neigh, wb_ref[...], preferred_element_type=jnp.float32))


def kernel(adj, inputs, W0, W1):
    n, d_in = inputs.shape
    dh = W0.shape[0]
    bm = 512
    x16 = inputs.astype(jnp.bfloat16)
    ones_blk = jnp.zeros((n, 128), jnp.bfloat16).at[:, 0].set(1.0)
    x16ext = jnp.concatenate([x16, ones_blk], axis=1)
    wa0, wb0 = (W0[:, :d_in].T.astype(jnp.bfloat16),
                W0[:, d_in:].T.astype(jnp.bfloat16))
    wa1, wb1 = (W1[:, :dh].T.astype(jnp.bfloat16),
                W1[:, dh:].T.astype(jnp.bfloat16))

    h, deg = pl.pallas_call(
        _layer0_body,
        grid=(n // bm,),
        in_specs=[
            pl.BlockSpec((bm, n), lambda i: (i, 0)),
            pl.BlockSpec((bm, d_in), lambda i: (i, 0)),
            pl.BlockSpec((n, d_in + 128), lambda i: (0, 0)),
            pl.BlockSpec((d_in, dh), lambda i: (0, 0)),
            pl.BlockSpec((d_in, dh), lambda i: (0, 0)),
        ],
        out_specs=[
            pl.BlockSpec((bm, dh), lambda i: (i, 0)),
            pl.BlockSpec((bm, 1), lambda i: (i, 0)),
        ],
        out_shape=[
            jax.ShapeDtypeStruct((n, dh), jnp.bfloat16),
            jax.ShapeDtypeStruct((n, 1), jnp.float32),
        ],
    )(adj, x16, x16ext, wa0, wb0)

    return pl.pallas_call(
        _layer1_body,
        grid=(n // bm,),
        in_specs=[
            pl.BlockSpec((bm, n), lambda i: (i, 0)),
            pl.BlockSpec((bm, dh), lambda i: (i, 0)),
            pl.BlockSpec((n, dh), lambda i: (0, 0)),
            pl.BlockSpec((dh, dh), lambda i: (0, 0)),
            pl.BlockSpec((dh, dh), lambda i: (0, 0)),
            pl.BlockSpec((bm, 1), lambda i: (i, 0)),
        ],
        out_specs=pl.BlockSpec((bm, dh), lambda i: (i, 0)),
        out_shape=jax.ShapeDtypeStruct((n, dh), jnp.float32),
    )(adj, h, h, wa1, wb1, deg)


# speedup vs baseline: 1.0069x; 1.0069x over previous
import functools

import jax
import jax.numpy as jnp
from jax.experimental import pallas as pl
from jax.experimental.pallas import tpu as pltpu


def _layer_body(apply_relu, first_layer, adj_hbm, xblk_ref, feats_ref,
                wa_ref, wb_ref, deg_ref, out_ref, odeg_ref, buf, sems):
    i = pl.program_id(0)
    ni = pl.num_programs(0)
    bm = buf.shape[1]

    def copy_in(blk, slot):
        return pltpu.make_async_copy(
            adj_hbm.at[pl.ds(blk * bm, bm), :], buf.at[slot], sems.at[slot])

    @pl.when(i == 0)
    def _():
        copy_in(0, 0).start()

    @pl.when(i + 1 < ni)
    def _():
        copy_in(i + 1, (i + 1) % 2).start()

    copy_in(i, i % 2).wait()
    a = buf[i % 2]
    p = jnp.dot(a.astype(jnp.bfloat16), feats_ref[...],
                preferred_element_type=jnp.float32)
    if first_layer:
        deg = jnp.sum(a, axis=1, keepdims=True) + 1.0
    else:
        deg = deg_ref[...]
    odeg_ref[...] = deg
    neigh = (p / deg).astype(jnp.bfloat16)
    out = (jnp.dot(xblk_ref[...], wa_ref[...], preferred_element_type=jnp.float32)
           + jnp.dot(neigh, wb_ref[...], preferred_element_type=jnp.float32))
    if apply_relu:
        out = jnp.maximum(out, 0.0)
    out_ref[...] = out.astype(out_ref.dtype)


def _sage_layer(adj, x16, wa, wb, deg, first_layer, apply_relu, out_dtype, bm):
    n, d = x16.shape
    dh = wa.shape[1]
    return pl.pallas_call(
        functools.partial(_layer_body, apply_relu, first_layer),
        grid=(n // bm,),
        in_specs=[
            pl.BlockSpec(memory_space=pltpu.MemorySpace.HBM),
            pl.BlockSpec((bm, d), lambda i: (i, 0)),
            pl.BlockSpec((n, d), lambda i: (0, 0)),
            pl.BlockSpec((d, dh), lambda i: (0, 0)),
            pl.BlockSpec((d, dh), lambda i: (0, 0)),
            pl.BlockSpec((bm, 1), lambda i: (i, 0)),
        ],
        out_specs=[
            pl.BlockSpec((bm, dh), lambda i: (i, 0)),
            pl.BlockSpec((bm, 1), lambda i: (i, 0)),
        ],
        out_shape=[
            jax.ShapeDtypeStruct((n, dh), out_dtype),
            jax.ShapeDtypeStruct((n, 1), jnp.float32),
        ],
        scratch_shapes=[
            pltpu.VMEM((2, bm, n), jnp.float32),
            pltpu.SemaphoreType.DMA((2,)),
        ],
        compiler_params=pltpu.CompilerParams(vmem_limit_bytes=100 * 1024 * 1024),
    )(adj, x16, x16, wa, wb, deg)


def kernel(adj, inputs, W0, W1):
    n, d_in = inputs.shape
    dh = W0.shape[0]
    x16 = inputs.astype(jnp.bfloat16)
    wa0, wb0 = (W0[:, :d_in].T.astype(jnp.bfloat16),
                W0[:, d_in:].T.astype(jnp.bfloat16))
    wa1, wb1 = (W1[:, :dh].T.astype(jnp.bfloat16),
                W1[:, dh:].T.astype(jnp.bfloat16))
    dummy_deg = jnp.ones((n, 1), jnp.float32)
    h, deg = _sage_layer(adj, x16, wa0, wb0, dummy_deg, first_layer=True,
                         apply_relu=True, out_dtype=jnp.bfloat16, bm=512)
    out, _ = _sage_layer(adj, h, wa1, wb1, deg, first_layer=False,
                         apply_relu=False, out_dtype=jnp.float32, bm=512)
    return out


# speedup vs baseline: 1.0446x; 1.0374x over previous
import functools

import jax
import jax.numpy as jnp
from jax.experimental import pallas as pl
from jax.experimental.pallas import tpu as pltpu


def _layer_body(apply_relu, first_layer, adj_ref, xblk_ref, feats_ref,
                wa_ref, wb_ref, deg_ref, out_ref, odeg_ref, z_scr):
    @pl.when(pl.program_id(0) == 0)
    def _():
        z_scr[...] = jnp.dot(feats_ref[...], wb_ref[...],
                             preferred_element_type=jnp.float32
                             ).astype(jnp.bfloat16)

    a = adj_ref[...]
    p = jnp.dot(a.astype(jnp.bfloat16), z_scr[...],
                preferred_element_type=jnp.float32)
    if first_layer:
        deg = jnp.sum(a, axis=1, keepdims=True) + 1.0
    else:
        deg = deg_ref[...]
    odeg_ref[...] = deg
    out = (jnp.dot(xblk_ref[...], wa_ref[...], preferred_element_type=jnp.float32)
           + p * (1.0 / deg))
    if apply_relu:
        out = jnp.maximum(out, 0.0)
    out_ref[...] = out.astype(out_ref.dtype)


def _sage_layer(adj, x16, wa, wb, deg, first_layer, apply_relu, out_dtype, bm):
    n, d = x16.shape
    dh = wa.shape[1]
    return pl.pallas_call(
        functools.partial(_layer_body, apply_relu, first_layer),
        grid=(n // bm,),
        in_specs=[
            pl.BlockSpec((bm, n), lambda i: (i, 0)),
            pl.BlockSpec((bm, d), lambda i: (i, 0)),
            pl.BlockSpec((n, d), lambda i: (0, 0)),
            pl.BlockSpec((d, dh), lambda i: (0, 0)),
            pl.BlockSpec((d, dh), lambda i: (0, 0)),
            pl.BlockSpec((bm, 1), lambda i: (i, 0)),
        ],
        out_specs=[
            pl.BlockSpec((bm, dh), lambda i: (i, 0)),
            pl.BlockSpec((bm, 1), lambda i: (i, 0)),
        ],
        out_shape=[
            jax.ShapeDtypeStruct((n, dh), out_dtype),
            jax.ShapeDtypeStruct((n, 1), jnp.float32),
        ],
        scratch_shapes=[
            pltpu.VMEM((n, dh), jnp.bfloat16),
        ],
    )(adj, x16, x16, wa, wb, deg)


def kernel(adj, inputs, W0, W1):
    n, d_in = inputs.shape
    dh = W0.shape[0]
    x16 = inputs.astype(jnp.bfloat16)
    wa0, wb0 = (W0[:, :d_in].T.astype(jnp.bfloat16),
                W0[:, d_in:].T.astype(jnp.bfloat16))
    wa1, wb1 = (W1[:, :dh].T.astype(jnp.bfloat16),
                W1[:, dh:].T.astype(jnp.bfloat16))
    dummy_deg = jnp.ones((n, 1), jnp.float32)
    h, deg = _sage_layer(adj, x16, wa0, wb0, dummy_deg, first_layer=True,
                         apply_relu=True, out_dtype=jnp.bfloat16, bm=512)
    out, _ = _sage_layer(adj, h, wa1, wb1, deg, first_layer=False,
                         apply_relu=False, out_dtype=jnp.float32, bm=512)
    return out


# speedup vs baseline: 1.1085x; 1.0612x over previous
import functools

import jax
import jax.numpy as jnp
from jax.experimental import pallas as pl
from jax.experimental.pallas import tpu as pltpu


def _fused_body(bm, adj_ref, xblk_ref, x16_ref, wa0_ref, wb0_ref, wa1_ref,
                wb1_ref, out_ref, z_scr, h_scr, deg_scr):
    l = pl.program_id(0)
    i = pl.program_id(1)

    @pl.when((l == 0) & (i == 0))
    def _():
        z_scr[...] = jnp.dot(x16_ref[...], wb0_ref[...],
                             preferred_element_type=jnp.float32
                             ).astype(jnp.bfloat16)

    @pl.when((l == 1) & (i == 0))
    def _():
        z_scr[...] = jnp.dot(h_scr[...], wb1_ref[...],
                             preferred_element_type=jnp.float32
                             ).astype(jnp.bfloat16)

    a = adj_ref[...]
    p = jnp.dot(a.astype(jnp.bfloat16), z_scr[...],
                preferred_element_type=jnp.float32)

    @pl.when(l == 0)
    def _():
        deg_scr[pl.ds(i * bm, bm), :] = jnp.sum(a, axis=1, keepdims=True) + 1.0

    deg = deg_scr[pl.ds(i * bm, bm), :]
    xin = jnp.where(l == 0, xblk_ref[...], h_scr[pl.ds(i * bm, bm), :])
    w = jnp.where(l == 0, wa0_ref[...], wa1_ref[...])
    out = jnp.dot(xin, w, preferred_element_type=jnp.float32) + p * (1.0 / deg)
    out = jnp.where(l == 0, jnp.maximum(out, 0.0), out)

    @pl.when(l == 0)
    def _():
        h_scr[pl.ds(i * bm, bm), :] = out.astype(jnp.bfloat16)

    out_ref[...] = out


def kernel(adj, inputs, W0, W1):
    n, d_in = inputs.shape
    dh = W0.shape[0]
    bm = 512
    x16 = inputs.astype(jnp.bfloat16)
    wa0, wb0 = (W0[:, :d_in].T.astype(jnp.bfloat16),
                W0[:, d_in:].T.astype(jnp.bfloat16))
    wa1, wb1 = (W1[:, :dh].T.astype(jnp.bfloat16),
                W1[:, dh:].T.astype(jnp.bfloat16))
    return pl.pallas_call(
        functools.partial(_fused_body, bm),
        grid=(2, n // bm),
        in_specs=[
            pl.BlockSpec((bm, n), lambda l, i: (i, 0)),
            pl.BlockSpec((bm, d_in), lambda l, i: ((1 - l) * i, 0)),
            pl.BlockSpec((n, d_in), lambda l, i: (0, 0)),
            pl.BlockSpec((d_in, dh), lambda l, i: (0, 0)),
            pl.BlockSpec((d_in, dh), lambda l, i: (0, 0)),
            pl.BlockSpec((dh, dh), lambda l, i: (0, 0)),
            pl.BlockSpec((dh, dh), lambda l, i: (0, 0)),
        ],
        out_specs=pl.BlockSpec((bm, dh), lambda l, i: (i, 0)),
        out_shape=jax.ShapeDtypeStruct((n, dh), jnp.float32),
        scratch_shapes=[
            pltpu.VMEM((n, dh), jnp.bfloat16),
            pltpu.VMEM((n, dh), jnp.bfloat16),
            pltpu.VMEM((n, 1), jnp.float32),
        ],
    )(adj, x16, x16, wa0, wb0, wa1, wb1)


# speedup vs baseline: 1.1936x; 1.0768x over previous
import functools

import jax
import jax.numpy as jnp
from jax.experimental import pallas as pl
from jax.experimental.pallas import tpu as pltpu


def _fused_body(bm, adj_ref, x16_ref, wa0_ref, wb0_ref, wa1_ref,
                wb1_ref, out_ref, z_scr, sx_scr, h_scr, deg_scr):
    l = pl.program_id(0)
    i = pl.program_id(1)

    @pl.when((l == 0) & (i == 0))
    def _():
        z_scr[...] = jnp.dot(x16_ref[...], wb0_ref[...],
                             preferred_element_type=jnp.float32
                             ).astype(jnp.bfloat16)
        sx_scr[...] = jnp.dot(x16_ref[...], wa0_ref[...],
                              preferred_element_type=jnp.float32
                              ).astype(jnp.bfloat16)

    @pl.when((l == 1) & (i == 0))
    def _():
        z_scr[...] = jnp.dot(h_scr[...], wb1_ref[...],
                             preferred_element_type=jnp.float32
                             ).astype(jnp.bfloat16)
        sx_scr[...] = jnp.dot(h_scr[...], wa1_ref[...],
                              preferred_element_type=jnp.float32
                              ).astype(jnp.bfloat16)

    a = adj_ref[...]
    p = jnp.dot(a.astype(jnp.bfloat16), z_scr[...],
                preferred_element_type=jnp.float32)

    @pl.when(l == 0)
    def _():
        deg_scr[pl.ds(i * bm, bm), :] = jnp.sum(a, axis=1, keepdims=True) + 1.0

    deg = deg_scr[pl.ds(i * bm, bm), :]
    out = sx_scr[pl.ds(i * bm, bm), :].astype(jnp.float32) + p * (1.0 / deg)
    out = jnp.where(l == 0, jnp.maximum(out, 0.0), out)

    @pl.when(l == 0)
    def _():
        h_scr[pl.ds(i * bm, bm), :] = out.astype(jnp.bfloat16)

    out_ref[...] = out


def kernel(adj, inputs, W0, W1):
    n, d_in = inputs.shape
    dh = W0.shape[0]
    bm = 1024
    x16 = inputs.astype(jnp.bfloat16)
    wa0, wb0 = (W0[:, :d_in].T.astype(jnp.bfloat16),
                W0[:, d_in:].T.astype(jnp.bfloat16))
    wa1, wb1 = (W1[:, :dh].T.astype(jnp.bfloat16),
                W1[:, dh:].T.astype(jnp.bfloat16))
    return pl.pallas_call(
        functools.partial(_fused_body, bm),
        grid=(2, n // bm),
        in_specs=[
            pl.BlockSpec((bm, n), lambda l, i: (i, 0)),
            pl.BlockSpec((n, d_in), lambda l, i: (0, 0)),
            pl.BlockSpec((d_in, dh), lambda l, i: (0, 0)),
            pl.BlockSpec((d_in, dh), lambda l, i: (0, 0)),
            pl.BlockSpec((dh, dh), lambda l, i: (0, 0)),
            pl.BlockSpec((dh, dh), lambda l, i: (0, 0)),
        ],
        out_specs=pl.BlockSpec((bm, dh), lambda l, i: (l * i, 0)),
        out_shape=jax.ShapeDtypeStruct((n, dh), jnp.float32),
        scratch_shapes=[
            pltpu.VMEM((n, dh), jnp.bfloat16),
            pltpu.VMEM((n, dh), jnp.bfloat16),
            pltpu.VMEM((n, dh), jnp.bfloat16),
            pltpu.VMEM((n, 1), jnp.float32),
        ],
        compiler_params=pltpu.CompilerParams(
            vmem_limit_bytes=100 * 1024 * 1024),
    )(adj, x16, wa0, wb0, wa1, wb1)
